# baseline (speedup 1.0000x reference)
import jax
import jax.numpy as jnp
from jax import lax
from jax.experimental import pallas as pl
from jax.experimental.pallas import tpu as pltpu
from jax.experimental.pallas import tpu_sc as plsc


def _sc_build_mt(idx_hbm, w_hbm, mt_hbm, idx_v, w_v, mt_v):
    cid = lax.axis_index("c")
    sid = lax.axis_index("s")

    @pl.when(jnp.logical_and(cid == 0, sid == 0))
    def _():
        pltpu.sync_copy(idx_hbm, idx_v)
        pltpu.sync_copy(w_hbm, w_v)
        zeros = jnp.zeros((16,), jnp.float32)
        for r in range(32):
            for c16 in range(2):
                mt_v[r, pl.ds(c16 * 16, 16)] = zeros
        wvec = w_v[...]
        wks = [wvec[k] for k in range(4)]
        for hc in range(2):
            idx_vecs = [idx_v[pl.ds(k * 32 + hc * 16, 16)] for k in range(4)]
            for r in range(32):
                acc = zeros
                for k in range(4):
                    acc = acc + jnp.where(idx_vecs[k] == r, wks[k], 0.0)
                mt_v[r, pl.ds(hc * 16, 16)] = acc
        pltpu.sync_copy(mt_v, mt_hbm)


def _build_mt_sc(idx_flat, w_pad):
    mesh = plsc.VectorSubcoreMesh(core_axis_name="c", subcore_axis_name="s")
    f = pl.kernel(
        _sc_build_mt,
        out_type=jax.ShapeDtypeStruct((32, 128), jnp.float32),
        mesh=mesh,
        scratch_types=[
            pltpu.VMEM((128,), jnp.int32),
            pltpu.VMEM((16,), jnp.float32),
            pltpu.VMEM((32, 128), jnp.float32),
        ],
    )
    return f(idx_flat, w_pad)


def _fused_kernel(mtw_ref, lam_ref, x_ref, o_ref, at_ref):
    i = pl.program_id(0)
    j = pl.program_id(1)

    @pl.when(jnp.logical_and(i == 0, j == 0))
    def _power():
        h = at_ref.shape[0]
        mt = mtw_ref[:, 0:h]
        mt5 = mt
        for _ in range(4):
            mt5 = jnp.dot(mt, mt5, preferred_element_type=jnp.float32)
        lam = lam_ref[0, 0]
        at_ref[...] = (lam * lam * lam * lam * lam) * mt5

    h, wblk, c = x_ref.shape
    x2 = x_ref[...].reshape(h, wblk * c)
    ob = jax.lax.dot_general(
        at_ref[...], x2, (((0,), (0,)), ((), ())),
        preferred_element_type=jnp.float32)
    o_ref[...] = ob.reshape(h, wblk, c)


def kernel(inputs, ind1, w1, lambda1):
    b, h, w, c = inputs.shape
    k_fan = ind1.shape[0]

    idx_flat = ind1[..., 0].astype(jnp.int32).reshape(k_fan * h)
    w_pad = jnp.pad(w1.reshape(k_fan).astype(jnp.float32), (0, 16 - k_fan))
    lam = lambda1.reshape(1, 1).astype(jnp.float32)

    mtw = _build_mt_sc(idx_flat, w_pad)

    wblk = 256
    x3 = inputs.reshape(b * h, w, c)
    out3 = pl.pallas_call(
        _fused_kernel,
        grid=(b, w // wblk),
        in_specs=[
            pl.BlockSpec(memory_space=pltpu.VMEM),
            pl.BlockSpec(memory_space=pltpu.SMEM),
            pl.BlockSpec((h, wblk, c), lambda i, j: (i, j, 0)),
        ],
        out_specs=pl.BlockSpec((h, wblk, c), lambda i, j: (i, j, 0)),
        out_shape=jax.ShapeDtypeStruct((b * h, w, c), jnp.float32),
        scratch_shapes=[pltpu.VMEM((h, h), jnp.float32)],
    )(mtw, lam, x3)

    return out3.reshape(b, h, w, c)


# speedup vs baseline: 2.2875x; 2.2875x over previous
import jax
import jax.numpy as jnp
from jax.experimental import pallas as pl
from jax.experimental.pallas import tpu as pltpu


def _fused_kernel(idx_ref, w_ref, lam_ref, x_ref, o_ref, at_ref):
    i = pl.program_id(0)
    j = pl.program_id(1)

    @pl.when(jnp.logical_and(i == 0, j == 0))
    def _build():
        h = at_ref.shape[0]
        k_fan = idx_ref.shape[0]
        row = jax.lax.broadcasted_iota(jnp.int32, (h, h), 0)
        mt = jnp.zeros((h, h), dtype=jnp.float32)
        for k in range(k_fan):
            hit = (row == idx_ref[k:k + 1, :]).astype(jnp.float32)
            mt = mt + w_ref[0, k] * hit
        mt5 = mt
        for _ in range(4):
            mt5 = jnp.dot(mt, mt5, preferred_element_type=jnp.float32)
        lam = lam_ref[0, 0]
        at_ref[...] = (lam * lam * lam * lam * lam) * mt5

    h, wblk, c = x_ref.shape
    x2 = x_ref[...].reshape(h, wblk * c)
    ob = jax.lax.dot_general(
        at_ref[...], x2, (((0,), (0,)), ((), ())),
        preferred_element_type=jnp.float32)
    o_ref[...] = ob.reshape(h, wblk, c)


def kernel(inputs, ind1, w1, lambda1):
    b, h, w, c = inputs.shape
    k_fan = ind1.shape[0]

    idx = ind1[..., 0].astype(jnp.int32)
    wv = w1.reshape(1, k_fan).astype(jnp.float32)
    lam = lambda1.reshape(1, 1).astype(jnp.float32)

    wblk = 256
    x3 = inputs.reshape(b * h, w, c)
    out3 = pl.pallas_call(
        _fused_kernel,
        grid=(b, w // wblk),
        in_specs=[
            pl.BlockSpec(memory_space=pltpu.VMEM),
            pl.BlockSpec(memory_space=pltpu.SMEM),
            pl.BlockSpec(memory_space=pltpu.SMEM),
            pl.BlockSpec((h, wblk, c), lambda i, j: (i, j, 0)),
        ],
        out_specs=pl.BlockSpec((h, wblk, c), lambda i, j: (i, j, 0)),
        out_shape=jax.ShapeDtypeStruct((b * h, w, c), jnp.float32),
        scratch_shapes=[pltpu.VMEM((h, h), jnp.float32)],
    )(idx, wv, lam, x3)

    return out3.reshape(b, h, w, c)
